# Initial kernel scaffold; baseline (speedup 1.0000x reference)
import jax
import jax.numpy as jnp
from jax.experimental import pallas as pl


def kernel(x, lut):
    raise NotImplementedError("write your pallas kernel here")


# speedup vs baseline: 1.5483x; 1.5483x over previous
import jax
import jax.numpy as jnp
from jax import lax
from jax.experimental import pallas as pl
from jax.experimental.pallas import tpu as pltpu
from jax.experimental.pallas import tpu_sc as plsc

D_MODEL = 1024
SCALE = 32.0
LANES = 16

NC, NS = 2, 16
NW = NC * NS
B_TOT = 4 * 4096
B_PER_W = B_TOT // NW
CHUNK = 32
NBUF = 2
NCHUNK = B_PER_W // CHUNK
VECS = CHUNK * D_MODEL // LANES


def _emb_body(idx_hbm, lut_hbm, out_hbm, idx_v, buf0, buf1, sem0, sem1):
    bufs = (buf0, buf1)
    sems = (sem0, sem1)
    wid = lax.axis_index("s") * NC + lax.axis_index("c")
    base = wid * B_PER_W

    pltpu.sync_copy(idx_hbm.at[pl.ds(base, B_PER_W)], idx_v)

    def start_gather(i):
        b = i % NBUF
        return pltpu.async_copy(
            lut_hbm.at[idx_v.at[pl.ds(i * CHUNK, CHUNK)]], bufs[b], sems[b])

    copies = [None] * NCHUNK
    for i in range(NBUF):
        copies[i] = start_gather(i)

    for i in range(NCHUNK):
        buf = bufs[i % NBUF]
        copies[i].wait()

        @plsc.parallel_loop(0, VECS, unroll=8)
        def _scale(j):
            r = lax.shift_right_logical(j, 6)
            c = (j & 63) * LANES
            buf[r, pl.ds(c, LANES)] = buf[r, pl.ds(c, LANES)] * SCALE

        pltpu.sync_copy(buf, out_hbm.at[pl.ds(base + i * CHUNK, CHUNK)])
        if i + NBUF < NCHUNK:
            copies[i + NBUF] = start_gather(i + NBUF)


def kernel(x, lut):
    idx = x.reshape(-1).astype(jnp.int32)
    out = pl.kernel(
        _emb_body,
        out_type=jax.ShapeDtypeStruct((B_TOT, D_MODEL), jnp.float32),
        mesh=plsc.VectorSubcoreMesh(core_axis_name="c", subcore_axis_name="s"),
        scratch_types=[
            pltpu.VMEM((B_PER_W,), jnp.int32),
            pltpu.VMEM((CHUNK, D_MODEL), jnp.float32),
            pltpu.VMEM((CHUNK, D_MODEL), jnp.float32),
            pltpu.SemaphoreType.DMA,
            pltpu.SemaphoreType.DMA,
        ],
    )(idx, lut)
    return out.reshape(x.shape + (D_MODEL,))


# speedup vs baseline: 1.5653x; 1.0110x over previous
import jax
import jax.numpy as jnp
from jax import lax
from jax.experimental import pallas as pl
from jax.experimental.pallas import tpu as pltpu
from jax.experimental.pallas import tpu_sc as plsc

D_MODEL = 1024
SCALE = 32.0
LANES = 16

NC, NS = 2, 16
NW = NC * NS
B_TOT = 4 * 4096
B_PER_W = B_TOT // NW
CHUNK = 32
NBUF = 3
NCHUNK = B_PER_W // CHUNK
VECS = CHUNK * D_MODEL // LANES


def _emb_body(idx_hbm, lut_hbm, out_hbm, idx_v,
              buf0, buf1, buf2, gsem0, gsem1, gsem2, ssem0, ssem1, ssem2):
    bufs = (buf0, buf1, buf2)
    gsems = (gsem0, gsem1, gsem2)
    ssems = (ssem0, ssem1, ssem2)
    wid = lax.axis_index("s") * NC + lax.axis_index("c")
    base = wid * B_PER_W

    pltpu.sync_copy(idx_hbm.at[pl.ds(base, B_PER_W)], idx_v)

    def start_gather(i):
        b = i % NBUF
        return pltpu.async_copy(
            lut_hbm.at[idx_v.at[pl.ds(i * CHUNK, CHUNK)]], bufs[b], gsems[b])

    copies = [None] * NCHUNK
    stores = [None] * NCHUNK
    for i in range(NBUF - 1):
        copies[i] = start_gather(i)

    store_waited = set()
    for i in range(NCHUNK):
        b = i % NBUF
        buf = bufs[b]
        copies[i].wait()

        @plsc.parallel_loop(0, VECS, unroll=8)
        def _scale(j):
            r = lax.shift_right_logical(j, 6)
            c = (j & 63) * LANES
            buf[r, pl.ds(c, LANES)] = buf[r, pl.ds(c, LANES)] * SCALE

        stores[i] = pltpu.async_copy(
            buf, out_hbm.at[pl.ds(base + i * CHUNK, CHUNK)], ssems[b])
        j = i + NBUF - 1
        if j < NCHUNK:
            if i >= 1:
                stores[i - 1].wait()
                store_waited.add(i - 1)
            copies[j] = start_gather(j)

    for i in range(NCHUNK):
        if i not in store_waited:
            stores[i].wait()


def kernel(x, lut):
    idx = x.reshape(-1).astype(jnp.int32)
    out = pl.kernel(
        _emb_body,
        out_type=jax.ShapeDtypeStruct((B_TOT, D_MODEL), jnp.float32),
        mesh=plsc.VectorSubcoreMesh(core_axis_name="c", subcore_axis_name="s"),
        scratch_types=[
            pltpu.VMEM((B_PER_W,), jnp.int32),
            pltpu.VMEM((CHUNK, D_MODEL), jnp.float32),
            pltpu.VMEM((CHUNK, D_MODEL), jnp.float32),
            pltpu.VMEM((CHUNK, D_MODEL), jnp.float32),
            pltpu.SemaphoreType.DMA,
            pltpu.SemaphoreType.DMA,
            pltpu.SemaphoreType.DMA,
            pltpu.SemaphoreType.DMA,
            pltpu.SemaphoreType.DMA,
            pltpu.SemaphoreType.DMA,
        ],
    )(idx, lut)
    return out.reshape(x.shape + (D_MODEL,))


# speedup vs baseline: 1.5938x; 1.0182x over previous
import jax
import jax.numpy as jnp
from jax import lax
from jax.experimental import pallas as pl
from jax.experimental.pallas import tpu as pltpu
from jax.experimental.pallas import tpu_sc as plsc

D_MODEL = 1024
SCALE = 32.0
LANES = 16

NC, NS = 2, 16
NW = NC * NS
B_TOT = 4 * 4096
B_PER_W = B_TOT // NW
CHUNK = 16
NBUF = 6
NCHUNK = B_PER_W // CHUNK
VECS = CHUNK * D_MODEL // LANES


def _emb_body(idx_hbm, lut_hbm, out_hbm, idx_v, *scratch):
    bufs = scratch[:NBUF]
    gsems = scratch[NBUF:2*NBUF]
    ssems = scratch[2*NBUF:3*NBUF]
    wid = lax.axis_index("s") * NC + lax.axis_index("c")
    base = wid * B_PER_W

    pltpu.sync_copy(idx_hbm.at[pl.ds(base, B_PER_W)], idx_v)

    def start_gather(i):
        b = i % NBUF
        return pltpu.async_copy(
            lut_hbm.at[idx_v.at[pl.ds(i * CHUNK, CHUNK)]], bufs[b], gsems[b])

    copies = [None] * NCHUNK
    stores = [None] * NCHUNK
    for i in range(NBUF - 1):
        copies[i] = start_gather(i)

    store_waited = set()
    for i in range(NCHUNK):
        b = i % NBUF
        buf = bufs[b]
        copies[i].wait()

        @plsc.parallel_loop(0, VECS, unroll=8)
        def _scale(j):
            r = lax.shift_right_logical(j, 6)
            c = (j & 63) * LANES
            buf[r, pl.ds(c, LANES)] = buf[r, pl.ds(c, LANES)] * SCALE

        stores[i] = pltpu.async_copy(
            buf, out_hbm.at[pl.ds(base + i * CHUNK, CHUNK)], ssems[b])
        j = i + NBUF - 1
        if j < NCHUNK:
            if i >= 1:
                stores[i - 1].wait()
                store_waited.add(i - 1)
            copies[j] = start_gather(j)

    for i in range(NCHUNK):
        if i not in store_waited:
            stores[i].wait()


def kernel(x, lut):
    idx = x.reshape(-1).astype(jnp.int32)
    out = pl.kernel(
        _emb_body,
        out_type=jax.ShapeDtypeStruct((B_TOT, D_MODEL), jnp.float32),
        mesh=plsc.VectorSubcoreMesh(core_axis_name="c", subcore_axis_name="s"),
        scratch_types=[
            pltpu.VMEM((B_PER_W,), jnp.int32),
        ] + [pltpu.VMEM((CHUNK, D_MODEL), jnp.float32) for _ in range(NBUF)]
          + [pltpu.SemaphoreType.DMA for _ in range(2 * NBUF)],
    )(idx, lut)
    return out.reshape(x.shape + (D_MODEL,))


# speedup vs baseline: 1.5962x; 1.0015x over previous
import jax
import jax.numpy as jnp
from jax import lax
from jax.experimental import pallas as pl
from jax.experimental.pallas import tpu as pltpu
from jax.experimental.pallas import tpu_sc as plsc

D_MODEL = 1024
SCALE = 32.0
LANES = 16

NC, NS = 2, 16
NW = NC * NS
B_TOT = 4 * 4096
B_PER_W = B_TOT // NW
CHUNK = 16
NBUF = 6
NCHUNK = B_PER_W // CHUNK
VECS = CHUNK * D_MODEL // LANES


def _emb_body(idx_hbm, lut_hbm, out_hbm, idx_v, *scratch):
    bufs = scratch[:NBUF]
    gsems = scratch[NBUF:2*NBUF]
    ssems = scratch[2*NBUF:3*NBUF]
    wid = lax.axis_index("s") * NC + lax.axis_index("c")
    base = wid * B_PER_W

    pltpu.sync_copy(
        idx_hbm.at[wid // (4096 // B_PER_W), pl.ds((wid % (4096 // B_PER_W)) * B_PER_W, B_PER_W)],
        idx_v)

    def start_gather(i):
        b = i % NBUF
        return pltpu.async_copy(
            lut_hbm.at[idx_v.at[pl.ds(i * CHUNK, CHUNK)]], bufs[b], gsems[b])

    copies = [None] * NCHUNK
    stores = [None] * NCHUNK
    for i in range(NBUF - 1):
        copies[i] = start_gather(i)

    store_waited = set()
    for i in range(NCHUNK):
        b = i % NBUF
        buf = bufs[b]
        copies[i].wait()

        @plsc.parallel_loop(0, VECS, unroll=8)
        def _scale(j):
            r = lax.shift_right_logical(j, 6)
            c = (j & 63) * LANES
            buf[r, pl.ds(c, LANES)] = buf[r, pl.ds(c, LANES)] * SCALE

        stores[i] = pltpu.async_copy(
            buf, out_hbm.at[pl.ds(base + i * CHUNK, CHUNK)], ssems[b])
        j = i + NBUF - 1
        if j < NCHUNK:
            if i >= 1:
                stores[i - 1].wait()
                store_waited.add(i - 1)
            copies[j] = start_gather(j)

    for i in range(NCHUNK):
        if i not in store_waited:
            stores[i].wait()


def kernel(x, lut):
    idx = x.astype(jnp.int32)
    out = pl.kernel(
        _emb_body,
        out_type=jax.ShapeDtypeStruct((B_TOT, D_MODEL), jnp.float32),
        mesh=plsc.VectorSubcoreMesh(core_axis_name="c", subcore_axis_name="s"),
        scratch_types=[
            pltpu.VMEM((B_PER_W,), jnp.int32),
        ] + [pltpu.VMEM((CHUNK, D_MODEL), jnp.float32) for _ in range(NBUF)]
          + [pltpu.SemaphoreType.DMA for _ in range(2 * NBUF)],
    )(idx, lut)
    return out.reshape(x.shape + (D_MODEL,))
